# baseline (speedup 1.0000x reference)
import functools

import jax
import jax.numpy as jnp
from jax import lax
from jax.experimental import pallas as pl
from jax.experimental.pallas import tpu as pltpu
from jax.experimental.pallas import tpu_sc as plsc

_ALPHA = 0.7
_K = 205
_N = 1024
_NW = 32
_ROWS = 64 * 768
_RPW = _ROWS // _NW
_GROUPS = _RPW // 16
_NBUCK = 512
_PAD = 1040
_HPAD = 528
_LO = -8.0
_SCALE = _NBUCK / 16.0
_INV = 1.0 / _SCALE
_RANK_TOP = float(_N - _K + 1)
_RANK_BOT = float(_K)


def _sc_body(x_hbm, out_hbm, xb0, xb1, hist, outv, sem0, sem1):
    wid = lax.axis_index("s") * 2 + lax.axis_index("c")
    row0 = wid * _RPW
    lanes = lax.iota(jnp.int32, 16)
    gbase = lanes * _PAD
    ones = jnp.ones((16,), jnp.float32)
    zeros = jnp.zeros((16,), jnp.float32)
    bufs = (xb0, xb1)
    sems = (sem0, sem1)

    def dma_group(g, buf, sem, wait):
        for r in range(16):
            cp = pltpu.make_async_copy(
                x_hbm.at[pl.ds((row0 + g * 16 + r) * _N, _N)],
                buf.at[pl.ds(r * _PAD, _N)],
                sem)
            if wait:
                cp.wait()
            else:
                cp.start()

    hbase = lanes * _HPAD

    @plsc.parallel_loop(0, _HPAD, unroll=8)
    def _(i):
        hist[pl.ds(i * 16, 16)] = zeros

    dma_group(0, xb0, sem0, False)
    dma_group(1, xb1, sem1, False)

    def group_body(g, buf, sem):
        dma_group(g, buf, sem, True)

        @plsc.parallel_loop(0, _N, unroll=8)
        def _(e):
            v = plsc.load_gather(buf, [gbase + e])
            bi = jnp.clip(v * _SCALE - (_LO * _SCALE), 0.0, _NBUCK - 1.0)
            plsc.addupdate_scatter(hist, [bi.astype(jnp.int32) + hbase],
                                   ones)

        @plsc.parallel_loop(0, _NBUCK, unroll=8,
                            carry=(zeros, zeros, zeros))
        def scan_out(i, carry):
            cum, btop, bbot = carry
            h = plsc.load_gather(hist, [hbase + i])
            plsc.store_scatter(hist, [hbase + i], zeros)
            cum = cum + h
            btop = btop + jnp.where(cum < _RANK_TOP, 1.0, 0.0)
            bbot = bbot + jnp.where(cum < _RANK_BOT, 1.0, 0.0)
            return cum, btop, bbot
        _, btop, bbot = scan_out
        t_top = _LO + btop * _INV
        t_bot = _LO + (bbot + 1.0) * _INV

        @plsc.parallel_loop(0, _N, unroll=8, carry=(zeros, zeros))
        def pass_b(e, carry):
            s1, s2 = carry
            v = plsc.load_gather(buf, [gbase + e])
            s1 = s1 + jnp.maximum(v - t_top, 0.0)
            s2 = s2 + jnp.maximum(t_bot - v, 0.0)
            return s1, s2
        s1, s2 = pass_b

        out = (t_top + s1 * (1.0 / _K)
               + _ALPHA * (t_bot - s2 * (1.0 / _K)))
        outv[pl.ds(g * 16, 16)] = out

    def outer(i, _):
        for b in range(2):
            g = i * 2 + b
            group_body(g, bufs[b], sems[b])

            @pl.when(g + 2 < _GROUPS)
            def _():
                dma_group(g + 2, bufs[b], sems[b], False)
        return 0
    lax.fori_loop(0, _GROUPS // 2, outer, 0)

    pltpu.sync_copy(outv, out_hbm.at[pl.ds(row0, _RPW)])


@jax.jit
def _wildcat_sc(flat):
    mesh = plsc.VectorSubcoreMesh(core_axis_name="c", subcore_axis_name="s")
    k = functools.partial(
        pl.kernel,
        mesh=mesh,
        compiler_params=pltpu.CompilerParams(use_tc_tiling_on_sc=False,
                                             needs_layout_passes=False),
        out_type=jax.ShapeDtypeStruct((_ROWS,), jnp.float32),
        scratch_types=[
            pltpu.VMEM((16 * _PAD,), jnp.float32),
            pltpu.VMEM((16 * _PAD,), jnp.float32),
            pltpu.VMEM((_HPAD * 16,), jnp.float32),
            pltpu.VMEM((_RPW,), jnp.float32),
            pltpu.SemaphoreType.DMA,
            pltpu.SemaphoreType.DMA,
        ],
    )(_sc_body)
    return k(flat)


def kernel(input):
    b, c, h, w = input.shape
    flat = input.reshape(b * c * h * w)
    return _wildcat_sc(flat).reshape(b, c)


# speedup vs baseline: 1.0241x; 1.0241x over previous
import functools

import jax
import jax.numpy as jnp
from jax import lax
from jax.experimental import pallas as pl
from jax.experimental.pallas import tpu as pltpu
from jax.experimental.pallas import tpu_sc as plsc

_ALPHA = 0.7
_K = 205
_N = 1024
_NW = 32
_ROWS = 64 * 768
_RPW = _ROWS // _NW
_GROUPS = _RPW // 16
_NBUCK = 512
_PAD = 1040
_HPAD = 528
_LO = -8.0
_SCALE = _NBUCK / 16.0
_INV = 1.0 / _SCALE
_RANK_TOP = float(_N - _K + 1)
_RANK_BOT = float(_K)


def _tree(xs):
    while len(xs) > 1:
        xs = [xs[i] + xs[i + 1] for i in range(0, len(xs) - 1, 2)] \
            + ([xs[-1]] if len(xs) % 2 else [])
    return xs[0]


def _sc_body(x_hbm, out_hbm, xb0, xb1, hist, outv, sem0, sem1):
    wid = lax.axis_index("s") * 2 + lax.axis_index("c")
    row0 = wid * _RPW
    lanes = lax.iota(jnp.int32, 16)
    gbase = lanes * _PAD
    ones = jnp.ones((16,), jnp.float32)
    zeros = jnp.zeros((16,), jnp.float32)
    bufs = (xb0, xb1)
    sems = (sem0, sem1)

    def dma_group(g, buf, sem, wait):
        for r in range(16):
            cp = pltpu.make_async_copy(
                x_hbm.at[pl.ds((row0 + g * 16 + r) * _N, _N)],
                buf.at[pl.ds(r * _PAD, _N)],
                sem)
            if wait:
                cp.wait()
            else:
                cp.start()

    @plsc.parallel_loop(0, _NBUCK, unroll=8)
    def _(i):
        hist[pl.ds(i * 16, 16)] = zeros

    dma_group(0, xb0, sem0, False)
    dma_group(1, xb1, sem1, False)

    def group_body(g, buf, sem):
        dma_group(g, buf, sem, True)

        @plsc.parallel_loop(0, _N, unroll=8)
        def _(e):
            v = plsc.load_gather(buf, [gbase + e])
            bi = jnp.clip(v * _SCALE - (_LO * _SCALE), 0.0, _NBUCK - 1.0)
            plsc.addupdate_scatter(hist, [bi.astype(jnp.int32) * 16 + lanes],
                                   ones)

        @plsc.parallel_loop(0, _NBUCK, step=8, carry=(zeros, zeros, zeros))
        def scan_out(i, carry):
            cum, btop, bbot = carry
            hs = []
            for u in range(8):
                hs.append(hist[pl.ds((i + u) * 16, 16)])
                hist[pl.ds((i + u) * 16, 16)] = zeros
            pre = list(hs)
            for d in (1, 2, 4):
                pre = [pre[u] + pre[u - d] if u >= d else pre[u]
                       for u in range(8)]
            cums = [cum + p for p in pre]
            dt = _tree([jnp.where(c < _RANK_TOP, 1.0, 0.0) for c in cums])
            db = _tree([jnp.where(c < _RANK_BOT, 1.0, 0.0) for c in cums])
            return cums[7], btop + dt, bbot + db
        _, btop, bbot = scan_out
        t_top = _LO + btop * _INV
        t_bot = _LO + (bbot + 1.0) * _INV

        @plsc.parallel_loop(0, _N, step=8, carry=(zeros, zeros))
        def pass_b(e, carry):
            s1, s2 = carry
            r1, r2 = [], []
            for u in range(8):
                v = plsc.load_gather(buf, [gbase + e + u])
                r1.append(jnp.maximum(v - t_top, 0.0))
                r2.append(jnp.maximum(t_bot - v, 0.0))
            return s1 + _tree(r1), s2 + _tree(r2)
        s1, s2 = pass_b

        out = (t_top + s1 * (1.0 / _K)
               + _ALPHA * (t_bot - s2 * (1.0 / _K)))
        outv[pl.ds(g * 16, 16)] = out

    def outer(i, _):
        for b in range(2):
            g = i * 2 + b
            group_body(g, bufs[b], sems[b])

            @pl.when(g + 2 < _GROUPS)
            def _():
                dma_group(g + 2, bufs[b], sems[b], False)
        return 0
    lax.fori_loop(0, _GROUPS // 2, outer, 0)

    pltpu.sync_copy(outv, out_hbm.at[pl.ds(row0, _RPW)])


@jax.jit
def _wildcat_sc(flat):
    mesh = plsc.VectorSubcoreMesh(core_axis_name="c", subcore_axis_name="s")
    k = functools.partial(
        pl.kernel,
        mesh=mesh,
        compiler_params=pltpu.CompilerParams(use_tc_tiling_on_sc=False,
                                             needs_layout_passes=False),
        out_type=jax.ShapeDtypeStruct((_ROWS,), jnp.float32),
        scratch_types=[
            pltpu.VMEM((16 * _PAD,), jnp.float32),
            pltpu.VMEM((16 * _PAD,), jnp.float32),
            pltpu.VMEM((_NBUCK * 16,), jnp.float32),
            pltpu.VMEM((_RPW,), jnp.float32),
            pltpu.SemaphoreType.DMA,
            pltpu.SemaphoreType.DMA,
        ],
    )(_sc_body)
    return k(flat)


def kernel(input):
    b, c, h, w = input.shape
    flat = input.reshape(b * c * h * w)
    return _wildcat_sc(flat).reshape(b, c)


# speedup vs baseline: 1.1226x; 1.0962x over previous
import functools

import jax
import jax.numpy as jnp
from jax import lax
from jax.experimental import pallas as pl
from jax.experimental.pallas import tpu as pltpu
from jax.experimental.pallas import tpu_sc as plsc

_ALPHA = 0.7
_K = 205
_N = 1024
_NW = 32
_ROWS = 64 * 768
_RPW = _ROWS // _NW
_GROUPS = _RPW // 16
_NBUCK = 512
_PAD = 1040
_HPAD = 528
_LO = -8.0
_SCALE = _NBUCK / 16.0
_INV = 1.0 / _SCALE
_RANK_TOP = float(_N - _K + 1)
_RANK_BOT = float(_K)


def _tree(xs):
    while len(xs) > 1:
        xs = [xs[i] + xs[i + 1] for i in range(0, len(xs) - 1, 2)] \
            + ([xs[-1]] if len(xs) % 2 else [])
    return xs[0]


def _sc_body(x_hbm, out_hbm, xb0, xb1, hist, outv, sem0, sem1):
    wid = lax.axis_index("s") * 2 + lax.axis_index("c")
    row0 = wid * _RPW
    lanes = lax.iota(jnp.int32, 16)

    def esplat(e):
        return jnp.full((16,), e, jnp.int32)
    ones = jnp.ones((16,), jnp.float32)
    zeros = jnp.zeros((16,), jnp.float32)
    bufs = (xb0, xb1)
    sems = (sem0, sem1)

    def dma_group(g, buf, sem, wait):
        cp = pltpu.make_async_copy(
            x_hbm.at[pl.ds(row0 + g * 16, 16)],
            buf.at[:, pl.ds(0, _N)],
            sem)
        if wait:
            cp.wait()
        else:
            cp.start()

    @plsc.parallel_loop(0, _NBUCK, unroll=8)
    def _(i):
        hist[pl.ds(i * 16, 16)] = zeros

    dma_group(0, xb0, sem0, False)
    dma_group(1, xb1, sem1, False)

    def group_body(g, buf, sem):
        dma_group(g, buf, sem, True)

        @plsc.parallel_loop(0, _N, unroll=8)
        def _(e):
            v = plsc.load_gather(buf, [lanes, esplat(e)])
            bi = jnp.clip(v * _SCALE - (_LO * _SCALE), 0.0, _NBUCK - 1.0)
            plsc.addupdate_scatter(hist, [bi.astype(jnp.int32) * 16 + lanes],
                                   ones)

        @plsc.parallel_loop(0, _NBUCK, step=8, carry=(zeros, zeros, zeros))
        def scan_out(i, carry):
            cum, btop, bbot = carry
            hs = []
            for u in range(8):
                hs.append(hist[pl.ds((i + u) * 16, 16)])
                hist[pl.ds((i + u) * 16, 16)] = zeros
            pre = list(hs)
            for d in (1, 2, 4):
                pre = [pre[u] + pre[u - d] if u >= d else pre[u]
                       for u in range(8)]
            cums = [cum + p for p in pre]
            dt = _tree([jnp.where(c < _RANK_TOP, 1.0, 0.0) for c in cums])
            db = _tree([jnp.where(c < _RANK_BOT, 1.0, 0.0) for c in cums])
            return cums[7], btop + dt, bbot + db
        _, btop, bbot = scan_out
        t_top = _LO + btop * _INV
        t_bot = _LO + (bbot + 1.0) * _INV

        @plsc.parallel_loop(0, _N, step=8, carry=(zeros, zeros))
        def pass_b(e, carry):
            s1, s2 = carry
            r1, r2 = [], []
            for u in range(8):
                v = plsc.load_gather(buf, [lanes, esplat(e + u)])
                r1.append(jnp.maximum(v - t_top, 0.0))
                r2.append(jnp.maximum(t_bot - v, 0.0))
            return s1 + _tree(r1), s2 + _tree(r2)
        s1, s2 = pass_b

        out = (t_top + s1 * (1.0 / _K)
               + _ALPHA * (t_bot - s2 * (1.0 / _K)))
        outv[pl.ds(g * 16, 16)] = out

    def outer(i, _):
        for b in range(2):
            g = i * 2 + b
            group_body(g, bufs[b], sems[b])

            @pl.when(g + 2 < _GROUPS)
            def _():
                dma_group(g + 2, bufs[b], sems[b], False)
        return 0
    lax.fori_loop(0, _GROUPS // 2, outer, 0)

    pltpu.sync_copy(outv, out_hbm.at[pl.ds(row0, _RPW)])


@jax.jit
def _wildcat_sc(flat):
    mesh = plsc.VectorSubcoreMesh(core_axis_name="c", subcore_axis_name="s")
    k = functools.partial(
        pl.kernel,
        mesh=mesh,
        compiler_params=pltpu.CompilerParams(use_tc_tiling_on_sc=False,
                                             needs_layout_passes=False),
        out_type=jax.ShapeDtypeStruct((_ROWS,), jnp.float32),
        scratch_types=[
            pltpu.VMEM((16, _PAD), jnp.float32),
            pltpu.VMEM((16, _PAD), jnp.float32),
            pltpu.VMEM((_NBUCK * 16,), jnp.float32),
            pltpu.VMEM((_RPW,), jnp.float32),
            pltpu.SemaphoreType.DMA,
            pltpu.SemaphoreType.DMA,
        ],
    )(_sc_body)
    return k(flat)


def kernel(input):
    b, c, h, w = input.shape
    flat = input.reshape(b * c, h * w)
    return _wildcat_sc(flat).reshape(b, c)
